# baseline (speedup 1.0000x reference)
import functools

import jax
import jax.numpy as jnp
from jax import lax
from jax.experimental import pallas as pl
from jax.experimental.pallas import tpu as pltpu
from jax.experimental.pallas import tpu_sc as plsc

_SC_NUM_CORES = 2
_SC_NUM_SUBCORES = 16
_SC_NUM_WORKERS = _SC_NUM_CORES * _SC_NUM_SUBCORES

_BM = 512
_BN = 2048
_LANES = 128
_NEG = -1e30


def _sc_gather(table, idx):
    B = idx.shape[0]
    V, E = table.shape
    assert B % (8 * _SC_NUM_WORKERS) == 0
    b_per_w = B // _SC_NUM_WORKERS

    mesh = plsc.VectorSubcoreMesh(core_axis_name="c", subcore_axis_name="s")

    @functools.partial(
        pl.kernel,
        mesh=mesh,
        out_type=jax.ShapeDtypeStruct((B, E), jnp.float32),
        scratch_types=[
            pltpu.VMEM((b_per_w,), jnp.int32),
            pltpu.VMEM((b_per_w, E), jnp.float32),
            pltpu.SemaphoreType.DMA,
        ],
        compiler_params=pltpu.CompilerParams(use_tc_tiling_on_sc=False),
    )
    def gather_kernel(table_hbm, idx_hbm, out_hbm, idx_v, rows_v, sem):
        wid = lax.axis_index("s") * _SC_NUM_CORES + lax.axis_index("c")
        base = wid * b_per_w
        pltpu.sync_copy(idx_hbm.at[pl.ds(base, b_per_w)], idx_v)
        pltpu.async_copy(table_hbm.at[idx_v], rows_v, sem).wait()
        pltpu.sync_copy(rows_v, out_hbm.at[pl.ds(base, b_per_w)])

    return gather_kernel(table, idx)


def _dot_nt(a, bm):
    return lax.dot_general(
        a, bm, (((1,), (1,)), ((), ())), preferred_element_type=jnp.float32)


def _fused_body(emb_ref, w_ref, out_ref, m_ref, s_ref, lse_ref, *, bn, nv):
    ph = pl.program_id(1)
    iv = pl.program_id(2)
    x = _dot_nt(emb_ref[...], w_ref[...])
    g = bn // _LANES
    xs = [lax.slice_in_dim(x, k * _LANES, (k + 1) * _LANES, axis=1)
          for k in range(g)]

    @pl.when(ph == 0)
    def _():
        cm = xs[0]
        for k in range(1, g):
            cm = jnp.maximum(cm, xs[k])
        m_prev = jnp.where(iv == 0, -jnp.inf, m_ref[...])
        s_prev = jnp.where(iv == 0, 0.0, s_ref[...])
        m_new = jnp.maximum(m_prev, cm)
        ssum = jnp.exp(xs[0] - m_new)
        for k in range(1, g):
            ssum = ssum + jnp.exp(xs[k] - m_new)
        s_new = s_prev * jnp.exp(m_prev - m_new) + ssum
        m_ref[...] = m_new
        s_ref[...] = s_new

        @pl.when(iv == nv - 1)
        def _():
            mtot = jnp.max(m_new, axis=1, keepdims=True)
            stot = jnp.sum(s_new * jnp.exp(m_new - mtot), axis=1,
                           keepdims=True)
            lse_ref[...] = jnp.broadcast_to(mtot + jnp.log(stot),
                                            lse_ref.shape)

    @pl.when(ph == 1)
    def _():
        lse = lse_ref[...]
        for k in range(g):
            out_ref[:, pl.ds(k * _LANES, _LANES)] = xs[k] - lse


def _split_bf16(x):
    hi = x.astype(jnp.bfloat16)
    lo = (x - hi.astype(jnp.float32)).astype(jnp.bfloat16)
    return hi, lo


def kernel(context_word, emb_table, W, b):
    B = context_word.shape[0]
    V, E = emb_table.shape
    bm, bn = _BM, _BN
    nb = B // bm
    nv = pl.cdiv(V, bn)
    vpad = nv * bn
    K = E + 4

    emb = _sc_gather(emb_table, context_word).astype(jnp.bfloat16)

    b_hi, b_lo = _split_bf16(b)
    ones_v = jnp.ones((V, 1), jnp.bfloat16)
    w_aug = jnp.concatenate(
        [W.astype(jnp.bfloat16), -ones_v, -ones_v,
         b_hi.reshape(V, 1), b_lo.reshape(V, 1)], axis=1)
    pad_row = jnp.zeros((1, K), jnp.bfloat16).at[0, E + 2].set(_NEG)
    w_aug = jnp.concatenate(
        [w_aug, jnp.broadcast_to(pad_row, (vpad - V, K))], axis=0)

    ones_b = jnp.ones((B, 1), jnp.bfloat16)
    zeros_b = jnp.zeros((B, 2), jnp.bfloat16)
    emb0 = jnp.concatenate([emb, zeros_b, ones_b, ones_b], axis=1)

    out = pl.pallas_call(
        functools.partial(_fused_body, bn=bn, nv=nv),
        grid=(nb, 2, nv),
        in_specs=[
            pl.BlockSpec((bm, K), lambda ib, ph, iv: (ib, 0)),
            pl.BlockSpec((bn, K), lambda ib, ph, iv: (iv, 0)),
        ],
        out_specs=pl.BlockSpec((bm, bn), lambda ib, ph, iv: (ib, iv * ph)),
        out_shape=jax.ShapeDtypeStruct((B, V), jnp.float32),
        scratch_shapes=[
            pltpu.VMEM((bm, _LANES), jnp.float32),
            pltpu.VMEM((bm, _LANES), jnp.float32),
            pltpu.VMEM((bm, _LANES), jnp.float32),
        ],
        compiler_params=pltpu.CompilerParams(
            dimension_semantics=("arbitrary", "arbitrary", "arbitrary"),
        ),
    )(emb0, w_aug)
    return out


# speedup vs baseline: 1.0003x; 1.0003x over previous
import functools

import jax
import jax.numpy as jnp
from jax import lax
from jax.experimental import pallas as pl
from jax.experimental.pallas import tpu as pltpu
from jax.experimental.pallas import tpu_sc as plsc

_SC_NUM_CORES = 2
_SC_NUM_SUBCORES = 16
_SC_NUM_WORKERS = _SC_NUM_CORES * _SC_NUM_SUBCORES

_BM = 512
_BN = 2048
_LANES = 128
_NEG = -1e30


def _sc_gather(table, idx):
    B = idx.shape[0]
    V, E = table.shape
    assert B % (8 * _SC_NUM_WORKERS) == 0
    b_per_w = B // _SC_NUM_WORKERS

    mesh = plsc.VectorSubcoreMesh(core_axis_name="c", subcore_axis_name="s")

    @functools.partial(
        pl.kernel,
        mesh=mesh,
        out_type=jax.ShapeDtypeStruct((B, E), jnp.float32),
        scratch_types=[
            pltpu.VMEM((b_per_w,), jnp.int32),
            pltpu.VMEM((b_per_w, E), jnp.float32),
            pltpu.SemaphoreType.DMA,
        ],
        compiler_params=pltpu.CompilerParams(use_tc_tiling_on_sc=False),
    )
    def gather_kernel(table_hbm, idx_hbm, out_hbm, idx_v, rows_v, sem):
        wid = lax.axis_index("s") * _SC_NUM_CORES + lax.axis_index("c")
        base = wid * b_per_w
        pltpu.sync_copy(idx_hbm.at[pl.ds(base, b_per_w)], idx_v)
        pltpu.async_copy(table_hbm.at[idx_v], rows_v, sem).wait()
        pltpu.sync_copy(rows_v, out_hbm.at[pl.ds(base, b_per_w)])

    return gather_kernel(table, idx)


def _dot_nt(a, bm):
    return lax.dot_general(
        a, bm, (((1,), (1,)), ((), ())), preferred_element_type=jnp.float32)


def _fused_body(emb_ref, w_ref, out_ref, m_ref, s_ref, lse_ref, *, bn, nv):
    ph = pl.program_id(1)
    iv = pl.program_id(2)
    x = _dot_nt(emb_ref[...], w_ref[...])
    g = bn // _LANES
    xs = [lax.slice_in_dim(x, k * _LANES, (k + 1) * _LANES, axis=1)
          for k in range(g)]

    @pl.when(ph == 0)
    def _():
        cm = xs[0]
        for k in range(1, g):
            cm = jnp.maximum(cm, xs[k])
        m_prev = jnp.where(iv == 0, -jnp.inf, m_ref[...])
        s_prev = jnp.where(iv == 0, 0.0, s_ref[...])
        m_new = jnp.maximum(m_prev, cm)
        ssum = jnp.exp(xs[0] - m_new)
        for k in range(1, g):
            ssum = ssum + jnp.exp(xs[k] - m_new)
        s_new = s_prev * jnp.exp(m_prev - m_new) + ssum
        m_ref[...] = m_new
        s_ref[...] = s_new

        @pl.when(iv == nv - 1)
        def _():
            mtot = jnp.max(m_new, axis=1, keepdims=True)
            stot = jnp.sum(s_new * jnp.exp(m_new - mtot), axis=1,
                           keepdims=True)
            lse_ref[...] = jnp.broadcast_to(mtot + jnp.log(stot),
                                            lse_ref.shape)

    @pl.when(ph == 1)
    def _():
        lse = lse_ref[...]
        for k in range(g):
            out_ref[:, pl.ds(k * _LANES, _LANES)] = xs[k] - lse


def _split_bf16(x):
    hi = x.astype(jnp.bfloat16)
    lo = (x - hi.astype(jnp.float32)).astype(jnp.bfloat16)
    return hi, lo


def kernel(context_word, emb_table, W, b):
    B = context_word.shape[0]
    V, E = emb_table.shape
    bm, bn = _BM, _BN
    nb = B // bm
    nv = pl.cdiv(V, bn)
    vpad = nv * bn
    K = E + 4

    emb = _sc_gather(emb_table, context_word).astype(jnp.bfloat16)

    b_hi, b_lo = _split_bf16(b)
    ones_v = jnp.ones((V, 1), jnp.bfloat16)
    w_aug = jnp.concatenate(
        [W.astype(jnp.bfloat16), -ones_v, -ones_v,
         b_hi.reshape(V, 1), b_lo.reshape(V, 1)], axis=1)
    pad_row = jnp.zeros((1, K), jnp.bfloat16).at[0, E + 2].set(_NEG)
    w_aug = jnp.concatenate(
        [w_aug, jnp.broadcast_to(pad_row, (vpad - V, K))], axis=0)

    ones_b = jnp.ones((B, 1), jnp.bfloat16)
    zeros_b = jnp.zeros((B, 2), jnp.bfloat16)
    emb0 = jnp.concatenate([emb, zeros_b, ones_b, ones_b], axis=1)

    out = pl.pallas_call(
        functools.partial(_fused_body, bn=bn, nv=nv),
        grid=(nb, 2, nv),
        in_specs=[
            pl.BlockSpec((bm, K), lambda ib, ph, iv: (ib, 0)),
            pl.BlockSpec((bn, K), lambda ib, ph, iv: (iv, 0)),
        ],
        out_specs=pl.BlockSpec((bm, bn), lambda ib, ph, iv: (ib, iv * ph)),
        out_shape=jax.ShapeDtypeStruct((B, V), jnp.float32),
        scratch_shapes=[
            pltpu.VMEM((bm, _LANES), jnp.float32),
            pltpu.VMEM((bm, _LANES), jnp.float32),
            pltpu.VMEM((bm, _LANES), jnp.float32),
        ],
        compiler_params=pltpu.CompilerParams(
            dimension_semantics=("parallel", "arbitrary", "arbitrary"),
        ),
    )(emb0, w_aug)
    return out


# speedup vs baseline: 1.0200x; 1.0197x over previous
import functools

import jax
import jax.numpy as jnp
from jax import lax
from jax.experimental import pallas as pl
from jax.experimental.pallas import tpu as pltpu
from jax.experimental.pallas import tpu_sc as plsc

_SC_NUM_CORES = 2
_SC_NUM_SUBCORES = 16
_SC_NUM_WORKERS = _SC_NUM_CORES * _SC_NUM_SUBCORES

_BM = 512
_BN = 2048
_LANES = 128
_NEG = -1e30


def _sc_gather(table, idx):
    B = idx.shape[0]
    V, E = table.shape
    assert B % (8 * _SC_NUM_WORKERS) == 0
    b_per_w = B // _SC_NUM_WORKERS

    mesh = plsc.VectorSubcoreMesh(core_axis_name="c", subcore_axis_name="s")

    @functools.partial(
        pl.kernel,
        mesh=mesh,
        out_type=jax.ShapeDtypeStruct((B, E), jnp.float32),
        scratch_types=[
            pltpu.VMEM((b_per_w,), jnp.int32),
            pltpu.VMEM((b_per_w, E), jnp.float32),
            pltpu.SemaphoreType.DMA,
        ],
        compiler_params=pltpu.CompilerParams(use_tc_tiling_on_sc=False),
    )
    def gather_kernel(table_hbm, idx_hbm, out_hbm, idx_v, rows_v, sem):
        wid = lax.axis_index("s") * _SC_NUM_CORES + lax.axis_index("c")
        base = wid * b_per_w
        pltpu.sync_copy(idx_hbm.at[pl.ds(base, b_per_w)], idx_v)
        pltpu.async_copy(table_hbm.at[idx_v], rows_v, sem).wait()
        pltpu.sync_copy(rows_v, out_hbm.at[pl.ds(base, b_per_w)])

    return gather_kernel(table, idx)


def _dot_nt(a, bm):
    return lax.dot_general(
        a, bm, (((1,), (1,)), ((), ())), preferred_element_type=jnp.float32)


def _fused_body(emb_ref, w_ref, out_ref, m_ref, s_ref, lse_ref, *, bn, nv):
    ph = pl.program_id(1)
    iv = pl.program_id(2)
    wt = w_ref[pl.ds(iv * bn, bn), :]
    x = _dot_nt(emb_ref[...], wt)
    g = bn // _LANES
    xs = [lax.slice_in_dim(x, k * _LANES, (k + 1) * _LANES, axis=1)
          for k in range(g)]

    @pl.when(ph == 0)
    def _():
        cm = xs[0]
        for k in range(1, g):
            cm = jnp.maximum(cm, xs[k])
        m_prev = jnp.where(iv == 0, -jnp.inf, m_ref[...])
        s_prev = jnp.where(iv == 0, 0.0, s_ref[...])
        m_new = jnp.maximum(m_prev, cm)
        ssum = jnp.exp(xs[0] - m_new)
        for k in range(1, g):
            ssum = ssum + jnp.exp(xs[k] - m_new)
        s_new = s_prev * jnp.exp(m_prev - m_new) + ssum
        m_ref[...] = m_new
        s_ref[...] = s_new

        @pl.when(iv == nv - 1)
        def _():
            mtot = jnp.max(m_new, axis=1, keepdims=True)
            stot = jnp.sum(s_new * jnp.exp(m_new - mtot), axis=1,
                           keepdims=True)
            lse_ref[...] = jnp.broadcast_to(mtot + jnp.log(stot),
                                            lse_ref.shape)

    @pl.when(ph == 1)
    def _():
        lse = lse_ref[...]
        for k in range(g):
            out_ref[:, pl.ds(k * _LANES, _LANES)] = xs[k] - lse


def _split_bf16(x):
    hi = x.astype(jnp.bfloat16)
    lo = (x - hi.astype(jnp.float32)).astype(jnp.bfloat16)
    return hi, lo


def kernel(context_word, emb_table, W, b):
    B = context_word.shape[0]
    V, E = emb_table.shape
    bm, bn = _BM, _BN
    nb = B // bm
    nv = pl.cdiv(V, bn)
    vpad = nv * bn
    K = E + 4

    emb = _sc_gather(emb_table, context_word).astype(jnp.bfloat16)

    b_hi, b_lo = _split_bf16(b)
    ones_v = jnp.ones((V, 1), jnp.bfloat16)
    w_aug = jnp.concatenate(
        [W.astype(jnp.bfloat16), -ones_v, -ones_v,
         b_hi.reshape(V, 1), b_lo.reshape(V, 1)], axis=1)
    pad_row = jnp.zeros((1, K), jnp.bfloat16).at[0, E + 2].set(_NEG)
    w_aug = jnp.concatenate(
        [w_aug, jnp.broadcast_to(pad_row, (vpad - V, K))], axis=0)

    ones_b = jnp.ones((B, 1), jnp.bfloat16)
    zeros_b = jnp.zeros((B, 2), jnp.bfloat16)
    emb0 = jnp.concatenate([emb, zeros_b, ones_b, ones_b], axis=1)

    out = pl.pallas_call(
        functools.partial(_fused_body, bn=bn, nv=nv),
        grid=(nb, 2, nv),
        in_specs=[
            pl.BlockSpec((bm, K), lambda ib, ph, iv: (ib, 0)),
            pl.BlockSpec((vpad, K), lambda ib, ph, iv: (0, 0)),
        ],
        out_specs=pl.BlockSpec((bm, bn), lambda ib, ph, iv: (ib, iv * ph)),
        out_shape=jax.ShapeDtypeStruct((B, V), jnp.float32),
        scratch_shapes=[
            pltpu.VMEM((bm, _LANES), jnp.float32),
            pltpu.VMEM((bm, _LANES), jnp.float32),
            pltpu.VMEM((bm, _LANES), jnp.float32),
        ],
        compiler_params=pltpu.CompilerParams(
            dimension_semantics=("parallel", "arbitrary", "arbitrary"),
        ),
    )(emb0, w_aug)
    return out


# speedup vs baseline: 1.0588x; 1.0381x over previous
import functools

import jax
import jax.numpy as jnp
from jax import lax
from jax.experimental import pallas as pl
from jax.experimental.pallas import tpu as pltpu
from jax.experimental.pallas import tpu_sc as plsc

_SC_NUM_CORES = 2
_SC_NUM_SUBCORES = 16
_SC_NUM_WORKERS = _SC_NUM_CORES * _SC_NUM_SUBCORES

_BM = 1024
_BN = 2048
_LANES = 128
_NEG = -1e30


def _sc_gather(table, idx):
    B = idx.shape[0]
    V, E = table.shape
    assert B % (8 * _SC_NUM_WORKERS) == 0
    b_per_w = B // _SC_NUM_WORKERS

    mesh = plsc.VectorSubcoreMesh(core_axis_name="c", subcore_axis_name="s")

    @functools.partial(
        pl.kernel,
        mesh=mesh,
        out_type=jax.ShapeDtypeStruct((B, E), jnp.float32),
        scratch_types=[
            pltpu.VMEM((b_per_w,), jnp.int32),
            pltpu.VMEM((b_per_w, E), jnp.float32),
            pltpu.SemaphoreType.DMA,
        ],
        compiler_params=pltpu.CompilerParams(use_tc_tiling_on_sc=False),
    )
    def gather_kernel(table_hbm, idx_hbm, out_hbm, idx_v, rows_v, sem):
        wid = lax.axis_index("s") * _SC_NUM_CORES + lax.axis_index("c")
        base = wid * b_per_w
        pltpu.sync_copy(idx_hbm.at[pl.ds(base, b_per_w)], idx_v)
        pltpu.async_copy(table_hbm.at[idx_v], rows_v, sem).wait()
        pltpu.sync_copy(rows_v, out_hbm.at[pl.ds(base, b_per_w)])

    return gather_kernel(table, idx)


def _dot_nt(a, bm):
    return lax.dot_general(
        a, bm, (((1,), (1,)), ((), ())), preferred_element_type=jnp.float32)


def _fused_body(emb_ref, w_ref, out_ref, m_ref, s_ref, lse_ref, *, bn, nv):
    ph = pl.program_id(1)
    iv = pl.program_id(2)
    wt = w_ref[pl.ds(iv * bn, bn), :]
    x = _dot_nt(emb_ref[...], wt)
    g = bn // _LANES
    xs = [lax.slice_in_dim(x, k * _LANES, (k + 1) * _LANES, axis=1)
          for k in range(g)]

    @pl.when(ph == 0)
    def _():
        cm = xs[0]
        for k in range(1, g):
            cm = jnp.maximum(cm, xs[k])
        m_prev = jnp.where(iv == 0, -jnp.inf, m_ref[...])
        s_prev = jnp.where(iv == 0, 0.0, s_ref[...])
        m_new = jnp.maximum(m_prev, cm)
        ssum = jnp.exp(xs[0] - m_new)
        for k in range(1, g):
            ssum = ssum + jnp.exp(xs[k] - m_new)
        s_new = s_prev * jnp.exp(m_prev - m_new) + ssum
        m_ref[...] = m_new
        s_ref[...] = s_new

        @pl.when(iv == nv - 1)
        def _():
            mtot = jnp.max(m_new, axis=1, keepdims=True)
            stot = jnp.sum(s_new * jnp.exp(m_new - mtot), axis=1,
                           keepdims=True)
            lse_ref[...] = jnp.broadcast_to(mtot + jnp.log(stot),
                                            lse_ref.shape)

    @pl.when(ph == 1)
    def _():
        lse = lse_ref[...]
        for k in range(g):
            out_ref[:, pl.ds(k * _LANES, _LANES)] = xs[k] - lse


def _split_bf16(x):
    hi = x.astype(jnp.bfloat16)
    lo = (x - hi.astype(jnp.float32)).astype(jnp.bfloat16)
    return hi, lo


def kernel(context_word, emb_table, W, b):
    B = context_word.shape[0]
    V, E = emb_table.shape
    bm, bn = _BM, _BN
    nb = B // bm
    nv = pl.cdiv(V, bn)
    vpad = nv * bn
    K = E + 4

    emb = _sc_gather(emb_table, context_word).astype(jnp.bfloat16)

    b_hi, b_lo = _split_bf16(b)
    ones_v = jnp.ones((V, 1), jnp.bfloat16)
    w_aug = jnp.concatenate(
        [W.astype(jnp.bfloat16), -ones_v, -ones_v,
         b_hi.reshape(V, 1), b_lo.reshape(V, 1)], axis=1)
    pad_row = jnp.zeros((1, K), jnp.bfloat16).at[0, E + 2].set(_NEG)
    w_aug = jnp.concatenate(
        [w_aug, jnp.broadcast_to(pad_row, (vpad - V, K))], axis=0)

    ones_b = jnp.ones((B, 1), jnp.bfloat16)
    zeros_b = jnp.zeros((B, 2), jnp.bfloat16)
    emb0 = jnp.concatenate([emb, zeros_b, ones_b, ones_b], axis=1)

    out = pl.pallas_call(
        functools.partial(_fused_body, bn=bn, nv=nv),
        grid=(nb, 2, nv),
        in_specs=[
            pl.BlockSpec((bm, K), lambda ib, ph, iv: (ib, 0)),
            pl.BlockSpec((vpad, K), lambda ib, ph, iv: (0, 0)),
        ],
        out_specs=pl.BlockSpec((bm, bn), lambda ib, ph, iv: (ib, iv * ph)),
        out_shape=jax.ShapeDtypeStruct((B, V), jnp.float32),
        scratch_shapes=[
            pltpu.VMEM((bm, _LANES), jnp.float32),
            pltpu.VMEM((bm, _LANES), jnp.float32),
            pltpu.VMEM((bm, _LANES), jnp.float32),
        ],
        compiler_params=pltpu.CompilerParams(
            dimension_semantics=("parallel", "arbitrary", "arbitrary"),
            vmem_limit_bytes=110 * 1024 * 1024,
        ),
    )(emb0, w_aug)
    return out


# speedup vs baseline: 1.1739x; 1.1087x over previous
import functools

import jax
import jax.numpy as jnp
from jax import lax
from jax.experimental import pallas as pl
from jax.experimental.pallas import tpu as pltpu
from jax.experimental.pallas import tpu_sc as plsc

_SC_NUM_CORES = 2
_SC_NUM_SUBCORES = 16
_SC_NUM_WORKERS = _SC_NUM_CORES * _SC_NUM_SUBCORES

_BM = 512
_BN = 2048
_LANES = 128
_NEG = -1e30


def _sc_gather(table, idx):
    B = idx.shape[0]
    V, E = table.shape
    assert B % (8 * _SC_NUM_WORKERS) == 0
    b_per_w = B // _SC_NUM_WORKERS

    mesh = plsc.VectorSubcoreMesh(core_axis_name="c", subcore_axis_name="s")

    @functools.partial(
        pl.kernel,
        mesh=mesh,
        out_type=jax.ShapeDtypeStruct((B, E), jnp.float32),
        scratch_types=[
            pltpu.VMEM((b_per_w,), jnp.int32),
            pltpu.VMEM((b_per_w, E), jnp.float32),
            pltpu.SemaphoreType.DMA,
        ],
        compiler_params=pltpu.CompilerParams(use_tc_tiling_on_sc=False),
    )
    def gather_kernel(table_hbm, idx_hbm, out_hbm, idx_v, rows_v, sem):
        wid = lax.axis_index("s") * _SC_NUM_CORES + lax.axis_index("c")
        base = wid * b_per_w
        pltpu.sync_copy(idx_hbm.at[pl.ds(base, b_per_w)], idx_v)
        pltpu.async_copy(table_hbm.at[idx_v], rows_v, sem).wait()
        pltpu.sync_copy(rows_v, out_hbm.at[pl.ds(base, b_per_w)])

    return gather_kernel(table, idx)


def _dot_nt(a, bm):
    return lax.dot_general(
        a, bm, (((1,), (1,)), ((), ())), preferred_element_type=jnp.float32)


def _body(emb_a_ref, emb_b_ref, w_ref, out_ref, m_ref, s_ref, lse_ref,
          *, bn, nv, nb):
    j = pl.program_id(0)
    iv = pl.program_id(1)
    wt = w_ref[pl.ds(iv * bn, bn), :]
    g = bn // _LANES

    @pl.when(j < nb)
    def _():
        x = _dot_nt(emb_a_ref[...], wt)
        xs = [lax.slice_in_dim(x, k * _LANES, (k + 1) * _LANES, axis=1)
              for k in range(g)]
        cm = xs[0]
        for k in range(1, g):
            cm = jnp.maximum(cm, xs[k])
        m_prev = jnp.where(iv == 0, -jnp.inf, m_ref[...])
        s_prev = jnp.where(iv == 0, 0.0, s_ref[...])
        m_new = jnp.maximum(m_prev, cm)
        ssum = jnp.exp(xs[0] - m_new)
        for k in range(1, g):
            ssum = ssum + jnp.exp(xs[k] - m_new)
        s_new = s_prev * jnp.exp(m_prev - m_new) + ssum
        m_ref[...] = m_new
        s_ref[...] = s_new

        @pl.when(iv == nv - 1)
        def _():
            mtot = jnp.max(m_new, axis=1, keepdims=True)
            stot = jnp.sum(s_new * jnp.exp(m_new - mtot), axis=1,
                           keepdims=True)
            lse = jnp.broadcast_to(mtot + jnp.log(stot),
                                   (m_new.shape[0], _LANES))
            lse_ref[pl.ds(lax.rem(j, 2), 1)] = lse[None]

    @pl.when(j >= 1)
    def _():
        y = _dot_nt(emb_b_ref[...], wt)
        lse3 = lse_ref[pl.ds(lax.rem(j + 1, 2), 1)]
        lse = jax.lax.reshape(lse3, (lse3.shape[1], _LANES))
        for k in range(g):
            yk = lax.slice_in_dim(y, k * _LANES, (k + 1) * _LANES, axis=1)
            out_ref[:, pl.ds(k * _LANES, _LANES)] = yk - lse


def _split_bf16(x):
    hi = x.astype(jnp.bfloat16)
    lo = (x - hi.astype(jnp.float32)).astype(jnp.bfloat16)
    return hi, lo


def kernel(context_word, emb_table, W, b):
    B = context_word.shape[0]
    V, E = emb_table.shape
    bm, bn = _BM, _BN
    nb = B // bm
    nv = pl.cdiv(V, bn)
    vpad = nv * bn
    K = E + 4

    emb = _sc_gather(emb_table, context_word).astype(jnp.bfloat16)

    b_hi, b_lo = _split_bf16(b)
    ones_v = jnp.ones((V, 1), jnp.bfloat16)
    w_aug = jnp.concatenate(
        [W.astype(jnp.bfloat16), -ones_v, -ones_v,
         b_hi.reshape(V, 1), b_lo.reshape(V, 1)], axis=1)
    pad_row = jnp.zeros((1, K), jnp.bfloat16).at[0, E + 2].set(_NEG)
    w_aug = jnp.concatenate(
        [w_aug, jnp.broadcast_to(pad_row, (vpad - V, K))], axis=0)

    ones_b = jnp.ones((B, 1), jnp.bfloat16)
    zeros_b = jnp.zeros((B, 2), jnp.bfloat16)
    emb0 = jnp.concatenate([emb, zeros_b, ones_b, ones_b], axis=1)

    out = pl.pallas_call(
        functools.partial(_body, bn=bn, nv=nv, nb=nb),
        grid=(nb + 1, nv),
        in_specs=[
            pl.BlockSpec((bm, K),
                         lambda j, iv, _n=nb: (jnp.minimum(j, _n - 1), 0)),
            pl.BlockSpec((bm, K), lambda j, iv: (jnp.maximum(j - 1, 0), 0)),
            pl.BlockSpec((vpad, K), lambda j, iv: (0, 0)),
        ],
        out_specs=pl.BlockSpec(
            (bm, bn),
            lambda j, iv: (jnp.maximum(j - 1, 0),
                           iv * jnp.minimum(j, 1))),
        out_shape=jax.ShapeDtypeStruct((B, V), jnp.float32),
        scratch_shapes=[
            pltpu.VMEM((bm, _LANES), jnp.float32),
            pltpu.VMEM((bm, _LANES), jnp.float32),
            pltpu.VMEM((2, bm, _LANES), jnp.float32),
        ],
        compiler_params=pltpu.CompilerParams(
            dimension_semantics=("arbitrary", "arbitrary"),
            vmem_limit_bytes=110 * 1024 * 1024,
        ),
    )(emb0, emb0, w_aug)
    return out


# speedup vs baseline: 1.2207x; 1.0399x over previous
import functools

import jax
import jax.numpy as jnp
from jax import lax
from jax.experimental import pallas as pl
from jax.experimental.pallas import tpu as pltpu
from jax.experimental.pallas import tpu_sc as plsc

_SC_NUM_CORES = 2
_SC_NUM_SUBCORES = 16
_SC_NUM_WORKERS = _SC_NUM_CORES * _SC_NUM_SUBCORES

_BN = 512
_LANES = 128


def _sc_gather(table, idx):
    B = idx.shape[0]
    V, E = table.shape
    assert B % (8 * _SC_NUM_WORKERS) == 0
    b_per_w = B // _SC_NUM_WORKERS

    mesh = plsc.VectorSubcoreMesh(core_axis_name="c", subcore_axis_name="s")

    @functools.partial(
        pl.kernel,
        mesh=mesh,
        out_type=jax.ShapeDtypeStruct((B, E), jnp.float32),
        scratch_types=[
            pltpu.VMEM((b_per_w,), jnp.int32),
            pltpu.VMEM((b_per_w, E), jnp.float32),
            pltpu.SemaphoreType.DMA,
        ],
        compiler_params=pltpu.CompilerParams(use_tc_tiling_on_sc=False),
    )
    def gather_kernel(table_hbm, idx_hbm, out_hbm, idx_v, rows_v, sem):
        wid = lax.axis_index("s") * _SC_NUM_CORES + lax.axis_index("c")
        base = wid * b_per_w
        pltpu.sync_copy(idx_hbm.at[pl.ds(base, b_per_w)], idx_v)
        pltpu.async_copy(table_hbm.at[idx_v], rows_v, sem).wait()
        pltpu.sync_copy(rows_v, out_hbm.at[pl.ds(base, b_per_w)])

    return gather_kernel(table, idx)


def _dot_nt(a, bm):
    return lax.dot_general(
        a, bm, (((1,), (1,)), ((), ())), preferred_element_type=jnp.float32)


def _lse_body(emb_ref, w_ref, b_ref, lse_ref, m_ref, s_ref, *, bn, nv):
    iv = pl.program_id(0)
    x = _dot_nt(emb_ref[...], w_ref[...]) + b_ref[...]
    g = bn // _LANES
    xs = [lax.slice_in_dim(x, k * _LANES, (k + 1) * _LANES, axis=1)
          for k in range(g)]
    cm = xs[0]
    for k in range(1, g):
        cm = jnp.maximum(cm, xs[k])
    m_prev = jnp.where(iv == 0, -jnp.inf, m_ref[...])
    s_prev = jnp.where(iv == 0, 0.0, s_ref[...])
    m_new = jnp.maximum(m_prev, cm)
    ssum = jnp.exp(xs[0] - m_new)
    for k in range(1, g):
        ssum = ssum + jnp.exp(xs[k] - m_new)
    s_new = s_prev * jnp.exp(m_prev - m_new) + ssum
    m_ref[...] = m_new
    s_ref[...] = s_new

    @pl.when(iv == nv - 1)
    def _():
        mtot = jnp.max(m_new, axis=1, keepdims=True)
        stot = jnp.sum(s_new * jnp.exp(m_new - mtot), axis=1, keepdims=True)
        lse = mtot + jnp.log(stot)
        lse_ref[...] = jnp.broadcast_to(lse, lse_ref.shape)


def _project_body(emb_ref, w_ref, b_ref, lse_ref, out_ref, *, bn):
    x = _dot_nt(emb_ref[...], w_ref[...]) + b_ref[...]
    lse = lse_ref[...]
    for k in range(bn // _LANES):
        xk = lax.slice_in_dim(x, k * _LANES, (k + 1) * _LANES, axis=1)
        out_ref[:, pl.ds(k * _LANES, _LANES)] = xk - lse


def kernel(context_word, emb_table, W, b):
    B = context_word.shape[0]
    V, E = emb_table.shape
    bn = _BN
    nv = pl.cdiv(V, bn)
    vpad = nv * bn

    emb = _sc_gather(emb_table, context_word).astype(jnp.bfloat16)
    wp = jnp.pad(W, ((0, vpad - V), (0, 0))).astype(jnp.bfloat16)
    bp = jnp.pad(b.reshape(1, V), ((0, 0), (0, vpad - V)),
                 constant_values=-jnp.inf)

    lse = pl.pallas_call(
        functools.partial(_lse_body, bn=bn, nv=nv),
        grid=(nv,),
        in_specs=[
            pl.BlockSpec((B, E), lambda iv: (0, 0)),
            pl.BlockSpec((bn, E), lambda iv: (iv, 0)),
            pl.BlockSpec((1, bn), lambda iv: (0, iv)),
        ],
        out_specs=pl.BlockSpec((B, _LANES), lambda iv: (0, 0)),
        out_shape=jax.ShapeDtypeStruct((B, _LANES), jnp.float32),
        scratch_shapes=[
            pltpu.VMEM((B, _LANES), jnp.float32),
            pltpu.VMEM((B, _LANES), jnp.float32),
        ],
        compiler_params=pltpu.CompilerParams(
            dimension_semantics=("arbitrary",),
        ),
    )(emb, wp, bp)

    out = pl.pallas_call(
        functools.partial(_project_body, bn=bn),
        grid=(nv,),
        in_specs=[
            pl.BlockSpec((B, E), lambda iv: (0, 0)),
            pl.BlockSpec((bn, E), lambda iv: (iv, 0)),
            pl.BlockSpec((1, bn), lambda iv: (0, iv)),
            pl.BlockSpec((B, _LANES), lambda iv: (0, 0)),
        ],
        out_specs=pl.BlockSpec((B, bn), lambda iv: (0, iv)),
        out_shape=jax.ShapeDtypeStruct((B, V), jnp.float32),
        compiler_params=pltpu.CompilerParams(
            dimension_semantics=("parallel",),
        ),
    )(emb, wp, bp, lse)
    return out
